# baseline (speedup 1.0000x reference)
import functools

import jax
import jax.numpy as jnp
from jax import lax
from jax.experimental import pallas as pl
from jax.experimental.pallas import tpu as pltpu
from jax.experimental.pallas import tpu_sc as plsc

B = 4
T = 8192
D = 1024
NC = 2
NS = 16
NW = NC * NS
LANES = 16

B_SC = B

ROWS_PER_W = T // NW
CH = 8
NT = ROWS_PER_W // CH


def _build_sc():
    mesh = plsc.VectorSubcoreMesh(core_axis_name="c", subcore_axis_name="s")

    scratch = (
        [pltpu.VMEM((CH, D), jnp.float32) for _ in range(2 * B_SC)]
        + [pltpu.VMEM((CH, D), jnp.float32) for _ in range(2)]
        + [pltpu.SemaphoreType.DMA for _ in range(2 * B_SC)]
        + [pltpu.SemaphoreType.DMA for _ in range(2 * B_SC)]
        + [pltpu.SemaphoreType.DMA for _ in range(2)]
    )

    @functools.partial(
        pl.kernel,
        mesh=mesh,
        out_type=jax.ShapeDtypeStruct((B_SC, T, D), jnp.float32),
        scratch_types=scratch,
        compiler_params=pltpu.CompilerParams(use_tc_tiling_on_sc=True),
    )
    def k(x_hbm, t_hbm, o_hbm, *s):
        nb = 2 * B_SC
        xb = s[0:nb]
        tb = s[nb:nb + 2]
        s_in = s[nb + 2:2 * nb + 2]
        s_out = s[2 * nb + 2:3 * nb + 2]
        s_t = s[3 * nb + 2:3 * nb + 4]

        wid = lax.axis_index("c") * NS + lax.axis_index("s")
        row_base = wid * ROWS_PER_W

        def rows(g):
            return pl.ds(pl.multiple_of(row_base + g * CH, CH), CH)

        def tbl_copy(g, p):
            return pltpu.make_async_copy(t_hbm.at[rows(g)], tb[p], s_t[p])

        def in_copy(g, b, p):
            return pltpu.make_async_copy(
                x_hbm.at[b, rows(g)], xb[p * B_SC + b], s_in[p * B_SC + b])

        def out_copy(g, b, p):
            return pltpu.make_async_copy(
                xb[p * B_SC + b], o_hbm.at[b, rows(g)], s_out[p * B_SC + b])

        tbl_copy(0, 0).start()
        for b in range(B_SC):
            in_copy(0, b, 0).start()

        def pair_body(g2, carry):
            for p in range(2):
                g = g2 * 2 + p
                q = 1 - p

                @pl.when(g + 1 < NT)
                def _():
                    tbl_copy(g + 1, q).start()

                tbl_copy(g, p).wait()

                for b in range(B_SC):
                    in_copy(g, b, p).wait()

                    @pl.when(g + 1 < NT)
                    def _():
                        @pl.when(g >= 1)
                        def _():
                            out_copy(g - 1, b, q).wait()

                        in_copy(g + 1, b, q).start()

                    xbuf = xb[p * B_SC + b]
                    tbuf = tb[p]

                    def add_body(m, c):
                        tc0 = m // CH
                        r = m % CH
                        for kk in range(128 // LANES):
                            sl = pl.ds(tc0 * 128 + kk * LANES, LANES)
                            plsc.addupdate(xbuf.at[r, sl], tbuf[r, sl])
                        return c

                    lax.fori_loop(0, (D // 128) * CH, add_body, 0, unroll=2)

                    out_copy(g, b, p).start()

            return carry

        lax.fori_loop(0, NT // 2, pair_body, 0)

        for b in range(B_SC):
            out_copy(NT - 1, b, 1).wait()

    return k


_sc_add = _build_sc()


@jax.jit
def kernel(x, table):
    return _sc_add(x, table)


# speedup vs baseline: 1.0035x; 1.0035x over previous
import functools

import jax
import jax.numpy as jnp
from jax import lax
from jax.experimental import pallas as pl
from jax.experimental.pallas import tpu as pltpu
from jax.experimental.pallas import tpu_sc as plsc

B = 4
T = 8192
D = 1024
NC = 2
NS = 16
NW = NC * NS
LANES = 16

B_SC = B

ROWS_PER_W = T // NW
CH = 8
NT = ROWS_PER_W // CH


def _build_sc():
    mesh = plsc.VectorSubcoreMesh(core_axis_name="c", subcore_axis_name="s")

    scratch = (
        [pltpu.VMEM((CH, D), jnp.float32) for _ in range(2 * B_SC)]
        + [pltpu.VMEM((CH, D), jnp.float32) for _ in range(2)]
        + [pltpu.SemaphoreType.DMA for _ in range(2 * B_SC)]
        + [pltpu.SemaphoreType.DMA for _ in range(2 * B_SC)]
        + [pltpu.SemaphoreType.DMA for _ in range(2)]
    )

    @functools.partial(
        pl.kernel,
        mesh=mesh,
        out_type=jax.ShapeDtypeStruct((B_SC, T, D), jnp.float32),
        scratch_types=scratch,
        compiler_params=pltpu.CompilerParams(use_tc_tiling_on_sc=True),
    )
    def k(x_hbm, t_hbm, o_hbm, *s):
        nb = 2 * B_SC
        xb = s[0:nb]
        tb = s[nb:nb + 2]
        s_in = s[nb + 2:2 * nb + 2]
        s_out = s[2 * nb + 2:3 * nb + 2]
        s_t = s[3 * nb + 2:3 * nb + 4]

        wid = lax.axis_index("s") * NC + lax.axis_index("c")
        row_base = wid * ROWS_PER_W

        def rows(g):
            return pl.ds(pl.multiple_of(row_base + g * CH, CH), CH)

        def tbl_copy(g, p):
            return pltpu.make_async_copy(t_hbm.at[rows(g)], tb[p], s_t[p])

        def in_copy(g, b, p):
            return pltpu.make_async_copy(
                x_hbm.at[b, rows(g)], xb[p * B_SC + b], s_in[p * B_SC + b])

        def out_copy(g, b, p):
            return pltpu.make_async_copy(
                xb[p * B_SC + b], o_hbm.at[b, rows(g)], s_out[p * B_SC + b])

        tbl_copy(0, 0).start()
        for b in range(B_SC):
            in_copy(0, b, 0).start()

        def pair_body(g2, carry):
            for p in range(2):
                g = g2 * 2 + p
                q = 1 - p

                @pl.when(g + 1 < NT)
                def _():
                    tbl_copy(g + 1, q).start()

                tbl_copy(g, p).wait()

                for b in range(B_SC):
                    in_copy(g, b, p).wait()

                    @pl.when(g + 1 < NT)
                    def _():
                        @pl.when(g >= 1)
                        def _():
                            out_copy(g - 1, b, q).wait()

                        in_copy(g + 1, b, q).start()

                    xbuf = xb[p * B_SC + b]
                    tbuf = tb[p]

                    def add_body(m, c):
                        tc0 = m // CH
                        r = m % CH
                        for kk in range(128 // LANES):
                            sl = pl.ds(tc0 * 128 + kk * LANES, LANES)
                            plsc.addupdate(xbuf.at[r, sl], tbuf[r, sl])
                        return c

                    lax.fori_loop(0, (D // 128) * CH, add_body, 0, unroll=2)

                    out_copy(g, b, p).start()

            return carry

        lax.fori_loop(0, NT // 2, pair_body, 0)

        for b in range(B_SC):
            out_copy(NT - 1, b, 1).wait()

    return k


_sc_add = _build_sc()


@jax.jit
def kernel(x, table):
    return _sc_add(x, table)
